# Initial kernel scaffold; baseline (speedup 1.0000x reference)
import jax
import jax.numpy as jnp
from jax.experimental import pallas as pl


def kernel(x, e, edge_index, edge_weight, halo_info, mask_send, mask_recv, buffer_send, buffer_recv, neighboring_procs, SIZE, eu_Ws, eu_bs, eu_g, eu_b, nu_Ws, nu_bs, nu_g, nu_b):
    raise NotImplementedError("write your pallas kernel here")


# speedup vs baseline: 3.5195x; 3.5195x over previous
import functools

import jax
import jax.numpy as jnp
from jax import lax
from jax.experimental import pallas as pl
from jax.experimental.pallas import tpu as pltpu
from jax.experimental.pallas import tpu_sc as plsc

_NC = 2
_NS = 16
_NW = _NC * _NS
_CH = 80


def _sc_gather(x, src3d, dst3d, E, N, C):
    ew = E // _NW
    nch = ew // _CH
    mesh = plsc.VectorSubcoreMesh(core_axis_name="c", subcore_axis_name="s")

    @functools.partial(
        pl.kernel, mesh=mesh,
        out_type=[jax.ShapeDtypeStruct((E, C), jnp.float32),
                  jax.ShapeDtypeStruct((E, C), jnp.float32)],
        scratch_types=[
            pltpu.VMEM((nch, _CH), jnp.int32),
            pltpu.VMEM((nch, _CH), jnp.int32),
            pltpu.VMEM((_CH, C), jnp.float32),
            pltpu.VMEM((_CH, C), jnp.float32),
            pltpu.SemaphoreType.DMA,
            pltpu.SemaphoreType.DMA,
        ],
    )
    def k(x_hbm, src_hbm, dst_hbm, xs_hbm, xd_hbm, si, di, bs, bd, s0, s1):
        wid = lax.axis_index("s") * _NC + lax.axis_index("c")
        row0 = wid * nch
        pltpu.sync_copy(src_hbm.at[wid], si)
        pltpu.sync_copy(dst_hbm.at[wid], di)

        def body(j, carry):
            off = pl.multiple_of((row0 + j) * _CH, _CH)
            c0 = pltpu.async_copy(x_hbm.at[si.at[j]], bs, s0)
            c1 = pltpu.async_copy(x_hbm.at[di.at[j]], bd, s1)
            c0.wait()
            c1.wait()
            pltpu.sync_copy(bs, xs_hbm.at[pl.ds(off, _CH)])
            pltpu.sync_copy(bd, xd_hbm.at[pl.ds(off, _CH)])
            return carry

        lax.fori_loop(0, nch, body, 0)

    return k(x, src3d, dst3d)


def _sc_scatter_add(e2, dst3d, zeros, E, N, C):
    ew = E // _NW
    nch = ew // _CH
    stripe = 640
    mesh = plsc.VectorSubcoreMesh(core_axis_name="c", subcore_axis_name="s")

    @functools.partial(
        pl.kernel, mesh=mesh,
        out_type=jax.ShapeDtypeStruct((_NC * N, C), jnp.float32),
        scratch_types=[
            pltpu.VMEM((nch, _CH), jnp.int32),
            pltpu.VMEM((_CH, C), jnp.float32),
            pltpu.VMEM_SHARED((N, C), jnp.float32),
        ],
    )
    def k(e_hbm, dst_hbm, z_hbm, out_hbm, di, buf, acc):
        c = lax.axis_index("c")
        s = lax.axis_index("s")
        wid = s * _NC + c
        row0 = wid * nch
        soff = pl.multiple_of(jnp.minimum(s * stripe, N - stripe), 16)
        pltpu.sync_copy(dst_hbm.at[wid], di)
        pltpu.sync_copy(z_hbm, acc.at[pl.ds(soff, stripe)])
        plsc.subcore_barrier()

        def body(j, carry):
            off = pl.multiple_of((row0 + j) * _CH, _CH)
            pltpu.sync_copy(e_hbm.at[pl.ds(off, _CH)], buf)
            pltpu.sync_copy(buf, acc.at[di.at[j]], add=True)
            return carry

        lax.fori_loop(0, nch, body, 0)
        plsc.subcore_barrier()
        pltpu.sync_copy(acc.at[pl.ds(soff, stripe)],
                        out_hbm.at[pl.ds(pl.multiple_of(c * N + soff, 16), stripe)])

    return k(e2, dst3d, zeros)


def _elu(h):
    return jnp.where(h > 0.0, h, jnp.exp(h) - 1.0)


def _layernorm(h, g, b):
    mu = jnp.mean(h, axis=-1, keepdims=True)
    hc = h - mu
    var = jnp.mean(hc * hc, axis=-1, keepdims=True)
    return hc / jnp.sqrt(var + 1e-5) * g + b


def _edge_mlp(xs, xd, e, ew2, Ws, bs, g, b, E, C):
    BE = 3200

    def body(xs_r, xd_r, e_r, ew_r, w1_r, w2_r, w3_r, w4_r,
             b1_r, b2_r, b3_r, b4_r, g_r, bb_r, out_r):
        w1 = w1_r[...]
        e0 = e_r[...]
        h = (jnp.dot(xs_r[...], w1[:C], preferred_element_type=jnp.float32)
             + jnp.dot(xd_r[...], w1[C:2 * C], preferred_element_type=jnp.float32)
             + jnp.dot(e0, w1[2 * C:], preferred_element_type=jnp.float32)
             + b1_r[...])
        h = _elu(h)
        h = _elu(jnp.dot(h, w2_r[...], preferred_element_type=jnp.float32) + b2_r[...])
        h = _elu(jnp.dot(h, w3_r[...], preferred_element_type=jnp.float32) + b3_r[...])
        h = jnp.dot(h, w4_r[...], preferred_element_type=jnp.float32) + b4_r[...]
        out_r[...] = (e0 + _layernorm(h, g_r[...], bb_r[...])) * ew_r[...]

    eb = lambda i: (i, 0)
    zb = lambda i: (0, 0)
    return pl.pallas_call(
        body,
        grid=(E // BE,),
        in_specs=[pl.BlockSpec((BE, C), eb)] * 3
        + [pl.BlockSpec((BE, 1), eb),
           pl.BlockSpec((3 * C, C), zb)]
        + [pl.BlockSpec((C, C), zb)] * 3
        + [pl.BlockSpec((1, C), zb)] * 6,
        out_specs=pl.BlockSpec((BE, C), eb),
        out_shape=jax.ShapeDtypeStruct((E, C), jnp.float32),
    )(xs, xd, e, ew2, Ws[0], Ws[1], Ws[2], Ws[3],
      bs[0], bs[1], bs[2], bs[3], g, b)


def _node_mlp(x, parts, Ws, bs, g, b, N, C):
    BN = 2000

    def body(x_r, p0_r, p1_r, w1_r, w2_r, w3_r, w4_r,
             b1_r, b2_r, b3_r, b4_r, g_r, bb_r, out_r):
        x0 = x_r[...]
        agg = p0_r[...] + p1_r[...]
        w1 = w1_r[...]
        h = (jnp.dot(x0, w1[:C], preferred_element_type=jnp.float32)
             + jnp.dot(agg, w1[C:], preferred_element_type=jnp.float32)
             + b1_r[...])
        h = _elu(h)
        h = _elu(jnp.dot(h, w2_r[...], preferred_element_type=jnp.float32) + b2_r[...])
        h = _elu(jnp.dot(h, w3_r[...], preferred_element_type=jnp.float32) + b3_r[...])
        h = jnp.dot(h, w4_r[...], preferred_element_type=jnp.float32) + b4_r[...]
        out_r[...] = x0 + _layernorm(h, g_r[...], bb_r[...])

    nb = lambda i: (i, 0)
    zb = lambda i: (0, 0)
    p0 = lax.slice_in_dim(parts, 0, N, axis=0)
    p1 = lax.slice_in_dim(parts, N, 2 * N, axis=0)
    return pl.pallas_call(
        body,
        grid=(N // BN,),
        in_specs=[pl.BlockSpec((BN, C), nb)] * 3
        + [pl.BlockSpec((2 * C, C), zb)]
        + [pl.BlockSpec((C, C), zb)] * 3
        + [pl.BlockSpec((1, C), zb)] * 6,
        out_specs=pl.BlockSpec((BN, C), nb),
        out_shape=jax.ShapeDtypeStruct((N, C), jnp.float32),
    )(x, p0, p1, Ws[0], Ws[1], Ws[2], Ws[3],
      bs[0], bs[1], bs[2], bs[3], g, b)


def kernel(x, e, edge_index, edge_weight, halo_info, mask_send, mask_recv,
           buffer_send, buffer_recv, neighboring_procs, SIZE,
           eu_Ws, eu_bs, eu_g, eu_b, nu_Ws, nu_bs, nu_g, nu_b):
    N, C = x.shape
    E = e.shape[0]

    nch = E // _NW // _CH
    src3d = edge_index[0].reshape(_NW, nch, _CH)
    dst3d = edge_index[1].reshape(_NW, nch, _CH)

    xs, xd = _sc_gather(x, src3d, dst3d, E, N, C)

    e_out = _edge_mlp(
        xs, xd, e, edge_weight.reshape(E, 1),
        eu_Ws, [bb.reshape(1, C) for bb in eu_bs],
        eu_g.reshape(1, C), eu_b.reshape(1, C), E, C)

    zeros = jnp.zeros((640, C), jnp.float32)
    parts = _sc_scatter_add(e_out, dst3d, zeros, E, N, C)

    x_out = _node_mlp(
        x, parts,
        nu_Ws, [bb.reshape(1, C) for bb in nu_bs],
        nu_g.reshape(1, C), nu_b.reshape(1, C), N, C)

    return (x_out, e_out)


# speedup vs baseline: 3.8867x; 1.1043x over previous
import functools

import jax
import jax.numpy as jnp
from jax import lax
from jax.experimental import pallas as pl
from jax.experimental.pallas import tpu as pltpu
from jax.experimental.pallas import tpu_sc as plsc

_NC = 2
_NS = 16
_NW = _NC * _NS
_CH = 80
_PIPE = 4


def _sc_gather(x, src3d, dst3d, E, N, C):
    ew = E // _NW
    nch = ew // _CH
    nfull = (nch - 1) // _PIPE
    mesh = plsc.VectorSubcoreMesh(core_axis_name="c", subcore_axis_name="s")

    @functools.partial(
        pl.kernel, mesh=mesh,
        out_type=[jax.ShapeDtypeStruct((E, C), jnp.float32),
                  jax.ShapeDtypeStruct((E, C), jnp.float32)],
        scratch_types=[
            pltpu.VMEM((nch, _CH), jnp.int32),
            pltpu.VMEM((nch, _CH), jnp.int32),
        ]
        + [pltpu.VMEM((_CH, C), jnp.float32)] * (2 * _PIPE)
        + [pltpu.SemaphoreType.DMA] * (2 * _PIPE),
    )
    def k(x_hbm, src_hbm, dst_hbm, xs_hbm, xd_hbm, si, di, *bufsem):
        bufs = bufsem[:2 * _PIPE]
        gsem = bufsem[2 * _PIPE:3 * _PIPE]
        wsem = bufsem[3 * _PIPE:]
        wid = lax.axis_index("s") * _NC + lax.axis_index("c")
        row0 = wid * nch
        pltpu.sync_copy(src_hbm.at[wid], si)
        pltpu.sync_copy(dst_hbm.at[wid], di)

        def body(k_, carry):
            j0 = k_ * _PIPE
            gd = []
            for t in range(_PIPE):
                j = j0 + t
                gd.append(pltpu.async_copy(x_hbm.at[si.at[j]], bufs[2 * t], gsem[t]))
                gd.append(pltpu.async_copy(x_hbm.at[di.at[j]], bufs[2 * t + 1], gsem[t]))
            wd = []
            for t in range(_PIPE):
                j = j0 + t
                off = pl.multiple_of((row0 + j) * _CH, _CH)
                gd[2 * t].wait()
                gd[2 * t + 1].wait()
                wd.append(pltpu.async_copy(bufs[2 * t], xs_hbm.at[pl.ds(off, _CH)], wsem[t]))
                wd.append(pltpu.async_copy(bufs[2 * t + 1], xd_hbm.at[pl.ds(off, _CH)], wsem[t]))
            for d in wd:
                d.wait()
            return carry

        lax.fori_loop(0, nfull, body, 0)

        for j_ in range(nfull * _PIPE, nch):
            off = pl.multiple_of((row0 + j_) * _CH, _CH)
            c0 = pltpu.async_copy(x_hbm.at[si.at[j_]], bufs[0], gsem[0])
            c1 = pltpu.async_copy(x_hbm.at[di.at[j_]], bufs[1], gsem[0])
            c0.wait()
            c1.wait()
            pltpu.sync_copy(bufs[0], xs_hbm.at[pl.ds(off, _CH)])
            pltpu.sync_copy(bufs[1], xd_hbm.at[pl.ds(off, _CH)])

    return k(x, src3d, dst3d)


def _sc_scatter_add(e2, dst3d, zeros, E, N, C):
    ew = E // _NW
    nch = ew // _CH
    npipe = 3
    nfull = (nch - 1) // npipe
    stripe = 640
    mesh = plsc.VectorSubcoreMesh(core_axis_name="c", subcore_axis_name="s")

    @functools.partial(
        pl.kernel, mesh=mesh,
        out_type=jax.ShapeDtypeStruct((_NC * N, C), jnp.float32),
        scratch_types=[
            pltpu.VMEM((nch, _CH), jnp.int32),
            pltpu.VMEM_SHARED((N, C), jnp.float32),
        ]
        + [pltpu.VMEM((_CH, C), jnp.float32)] * npipe
        + [pltpu.SemaphoreType.DMA] * (2 * npipe),
    )
    def k(e_hbm, dst_hbm, z_hbm, out_hbm, di, acc, *bufsem):
        bufs = bufsem[:npipe]
        lsem = bufsem[npipe:2 * npipe]
        ssem = bufsem[2 * npipe:]
        c = lax.axis_index("c")
        s = lax.axis_index("s")
        wid = s * _NC + c
        row0 = wid * nch
        soff = pl.multiple_of(jnp.minimum(s * stripe, N - stripe), 16)
        pltpu.sync_copy(dst_hbm.at[wid], di)
        pltpu.sync_copy(z_hbm, acc.at[pl.ds(soff, stripe)])
        plsc.subcore_barrier()

        def body(k_, carry):
            j0 = k_ * npipe
            ld = []
            for t in range(npipe):
                off = pl.multiple_of((row0 + j0 + t) * _CH, _CH)
                ld.append(pltpu.async_copy(e_hbm.at[pl.ds(off, _CH)], bufs[t], lsem[t]))
            sd = []
            for t in range(npipe):
                ld[t].wait()
                sd.append(pltpu.async_copy(bufs[t], acc.at[di.at[j0 + t]], ssem[t],
                                           add=True))
            for d in sd:
                d.wait()
            return carry

        lax.fori_loop(0, nfull, body, 0)

        for j_ in range(nfull * npipe, nch):
            off = pl.multiple_of((row0 + j_) * _CH, _CH)
            pltpu.sync_copy(e_hbm.at[pl.ds(off, _CH)], bufs[0])
            pltpu.sync_copy(bufs[0], acc.at[di.at[j_]], add=True)

        plsc.subcore_barrier()
        pltpu.sync_copy(acc.at[pl.ds(soff, stripe)],
                        out_hbm.at[pl.ds(pl.multiple_of(c * N + soff, 16), stripe)])

    return k(e2, dst3d, zeros)


def _elu(h):
    return jnp.where(h > 0.0, h, jnp.exp(h) - 1.0)


def _layernorm(h, g, b):
    mu = jnp.mean(h, axis=-1, keepdims=True)
    hc = h - mu
    var = jnp.mean(hc * hc, axis=-1, keepdims=True)
    return hc / jnp.sqrt(var + 1e-5) * g + b


def _edge_mlp(xs, xd, e, ew2, Ws, bs, g, b, E, C):
    BE = 3200

    def body(xs_r, xd_r, e_r, ew_r, w1_r, w2_r, w3_r, w4_r,
             b1_r, b2_r, b3_r, b4_r, g_r, bb_r, out_r):
        w1 = w1_r[...]
        e0 = e_r[...]
        h = (jnp.dot(xs_r[...].astype(jnp.bfloat16), w1[:C],
                     preferred_element_type=jnp.float32)
             + jnp.dot(xd_r[...].astype(jnp.bfloat16), w1[C:2 * C],
                       preferred_element_type=jnp.float32)
             + jnp.dot(e0.astype(jnp.bfloat16), w1[2 * C:],
                       preferred_element_type=jnp.float32)
             + b1_r[...])
        h = _elu(h).astype(jnp.bfloat16)
        h = _elu(jnp.dot(h, w2_r[...], preferred_element_type=jnp.float32)
                 + b2_r[...]).astype(jnp.bfloat16)
        h = _elu(jnp.dot(h, w3_r[...], preferred_element_type=jnp.float32)
                 + b3_r[...]).astype(jnp.bfloat16)
        h = jnp.dot(h, w4_r[...], preferred_element_type=jnp.float32) + b4_r[...]
        out_r[...] = (e0 + _layernorm(h, g_r[...], bb_r[...])) * ew_r[...]

    eb = lambda i: (i, 0)
    zb = lambda i: (0, 0)
    return pl.pallas_call(
        body,
        grid=(E // BE,),
        in_specs=[pl.BlockSpec((BE, C), eb)] * 3
        + [pl.BlockSpec((BE, 1), eb),
           pl.BlockSpec((3 * C, C), zb)]
        + [pl.BlockSpec((C, C), zb)] * 3
        + [pl.BlockSpec((1, C), zb)] * 6,
        out_specs=pl.BlockSpec((BE, C), eb),
        out_shape=jax.ShapeDtypeStruct((E, C), jnp.float32),
    )(xs, xd, e, ew2, Ws[0], Ws[1], Ws[2], Ws[3],
      bs[0], bs[1], bs[2], bs[3], g, b)


def _node_mlp(x, parts, Ws, bs, g, b, N, C):
    BN = 2000

    def body(x_r, p0_r, p1_r, w1_r, w2_r, w3_r, w4_r,
             b1_r, b2_r, b3_r, b4_r, g_r, bb_r, out_r):
        x0 = x_r[...]
        agg = p0_r[...] + p1_r[...]
        w1 = w1_r[...]
        h = (jnp.dot(x0, w1[:C], preferred_element_type=jnp.float32)
             + jnp.dot(agg, w1[C:], preferred_element_type=jnp.float32)
             + b1_r[...])
        h = _elu(h)
        h = _elu(jnp.dot(h, w2_r[...], preferred_element_type=jnp.float32) + b2_r[...])
        h = _elu(jnp.dot(h, w3_r[...], preferred_element_type=jnp.float32) + b3_r[...])
        h = jnp.dot(h, w4_r[...], preferred_element_type=jnp.float32) + b4_r[...]
        out_r[...] = x0 + _layernorm(h, g_r[...], bb_r[...])

    nb = lambda i: (i, 0)
    zb = lambda i: (0, 0)
    p0 = lax.slice_in_dim(parts, 0, N, axis=0)
    p1 = lax.slice_in_dim(parts, N, 2 * N, axis=0)
    return pl.pallas_call(
        body,
        grid=(N // BN,),
        in_specs=[pl.BlockSpec((BN, C), nb)] * 3
        + [pl.BlockSpec((2 * C, C), zb)]
        + [pl.BlockSpec((C, C), zb)] * 3
        + [pl.BlockSpec((1, C), zb)] * 6,
        out_specs=pl.BlockSpec((BN, C), nb),
        out_shape=jax.ShapeDtypeStruct((N, C), jnp.float32),
    )(x, p0, p1, Ws[0], Ws[1], Ws[2], Ws[3],
      bs[0], bs[1], bs[2], bs[3], g, b)


def kernel(x, e, edge_index, edge_weight, halo_info, mask_send, mask_recv,
           buffer_send, buffer_recv, neighboring_procs, SIZE,
           eu_Ws, eu_bs, eu_g, eu_b, nu_Ws, nu_bs, nu_g, nu_b):
    N, C = x.shape
    E = e.shape[0]

    nch = E // _NW // _CH
    src3d = edge_index[0].reshape(_NW, nch, _CH)
    dst3d = edge_index[1].reshape(_NW, nch, _CH)

    xs, xd = _sc_gather(x, src3d, dst3d, E, N, C)

    e_out = _edge_mlp(
        xs, xd, e, edge_weight.reshape(E, 1),
        [w.astype(jnp.bfloat16) for w in eu_Ws],
        [bb.reshape(1, C) for bb in eu_bs],
        eu_g.reshape(1, C), eu_b.reshape(1, C), E, C)

    zeros = jnp.zeros((640, C), jnp.float32)
    parts = _sc_scatter_add(e_out, dst3d, zeros, E, N, C)

    x_out = _node_mlp(
        x, parts,
        nu_Ws, [bb.reshape(1, C) for bb in nu_bs],
        nu_g.reshape(1, C), nu_b.reshape(1, C), N, C)

    return (x_out, e_out)


# speedup vs baseline: 4.4194x; 1.1371x over previous
import functools

import jax
import jax.numpy as jnp
from jax import lax
from jax.experimental import pallas as pl
from jax.experimental.pallas import tpu as pltpu
from jax.experimental.pallas import tpu_sc as plsc

_NC = 2
_NS = 16
_NW = _NC * _NS
_CH = 80
_PIPE = 4


def _sc_gather(x, src3d, dst3d, E, N, C):
    ew = E // _NW
    chg = 40
    nsup = 5
    nch = ew // chg
    schn = nch // nsup
    npipe = 3
    nfull = schn // npipe
    ntail = schn - nfull * npipe
    stripe = 640
    mesh = plsc.VectorSubcoreMesh(core_axis_name="c", subcore_axis_name="s")

    @functools.partial(
        pl.kernel, mesh=mesh,
        out_type=[jax.ShapeDtypeStruct((E, C), jnp.float32),
                  jax.ShapeDtypeStruct((E, C), jnp.float32)],
        scratch_types=[
            pltpu.VMEM((schn, chg), jnp.int32),
            pltpu.VMEM((schn, chg), jnp.int32),
            pltpu.VMEM_SHARED((N, C), jnp.float32),
        ]
        + [pltpu.VMEM((chg, C), jnp.float32)] * (2 * npipe)
        + [pltpu.SemaphoreType.DMA] * (2 * npipe),
    )
    def k(x_hbm, src_hbm, dst_hbm, xs_hbm, xd_hbm, si, di, xspm, *bufsem):
        bufs = bufsem[:2 * npipe]
        gsem = bufsem[2 * npipe:3 * npipe]
        wsem = bufsem[3 * npipe:]
        s = lax.axis_index("s")
        wid = s * _NC + lax.axis_index("c")
        row0 = wid * nch
        soff = pl.multiple_of(jnp.minimum(s * stripe, N - stripe), 16)
        pltpu.sync_copy(x_hbm.at[pl.ds(soff, stripe)], xspm.at[pl.ds(soff, stripe)])
        plsc.subcore_barrier()

        def sup_body(sup, carry):
            pltpu.sync_copy(src_hbm.at[wid, sup], si)
            pltpu.sync_copy(dst_hbm.at[wid, sup], di)
            base = row0 + sup * schn

            def body(k_, carry2):
                j0 = k_ * npipe
                gd = []
                for t in range(npipe):
                    j = j0 + t
                    gd.append(pltpu.async_copy(xspm.at[si.at[j]], bufs[2 * t], gsem[t]))
                    gd.append(pltpu.async_copy(xspm.at[di.at[j]], bufs[2 * t + 1], gsem[t]))
                wd = []
                for t in range(npipe):
                    j = j0 + t
                    off = pl.multiple_of((base + j) * chg, chg)
                    gd[2 * t].wait()
                    gd[2 * t + 1].wait()
                    wd.append(pltpu.async_copy(bufs[2 * t], xs_hbm.at[pl.ds(off, chg)], wsem[t]))
                    wd.append(pltpu.async_copy(bufs[2 * t + 1], xd_hbm.at[pl.ds(off, chg)], wsem[t]))
                for d in wd:
                    d.wait()
                return carry2

            lax.fori_loop(0, nfull, body, 0)

            for j_ in range(nfull * npipe, schn):
                off = pl.multiple_of((base + j_) * chg, chg)
                c0 = pltpu.async_copy(xspm.at[si.at[j_]], bufs[0], gsem[0])
                c1 = pltpu.async_copy(xspm.at[di.at[j_]], bufs[1], gsem[0])
                c0.wait()
                c1.wait()
                pltpu.sync_copy(bufs[0], xs_hbm.at[pl.ds(off, chg)])
                pltpu.sync_copy(bufs[1], xd_hbm.at[pl.ds(off, chg)])
            return carry

        lax.fori_loop(0, nsup, sup_body, 0)

    return k(x, src3d, dst3d)


def _sc_scatter_add(e2, dst3d, zeros, E, N, C):
    ew = E // _NW
    nch = ew // _CH
    npipe = 3
    nfull = (nch - 1) // npipe
    stripe = 640
    mesh = plsc.VectorSubcoreMesh(core_axis_name="c", subcore_axis_name="s")

    @functools.partial(
        pl.kernel, mesh=mesh,
        out_type=jax.ShapeDtypeStruct((_NC * N, C), jnp.float32),
        scratch_types=[
            pltpu.VMEM((nch, _CH), jnp.int32),
            pltpu.VMEM_SHARED((N, C), jnp.float32),
        ]
        + [pltpu.VMEM((_CH, C), jnp.float32)] * npipe
        + [pltpu.SemaphoreType.DMA] * (2 * npipe),
    )
    def k(e_hbm, dst_hbm, z_hbm, out_hbm, di, acc, *bufsem):
        bufs = bufsem[:npipe]
        lsem = bufsem[npipe:2 * npipe]
        ssem = bufsem[2 * npipe:]
        c = lax.axis_index("c")
        s = lax.axis_index("s")
        wid = s * _NC + c
        row0 = wid * nch
        soff = pl.multiple_of(jnp.minimum(s * stripe, N - stripe), 16)
        pltpu.sync_copy(dst_hbm.at[wid], di)
        pltpu.sync_copy(z_hbm, acc.at[pl.ds(soff, stripe)])
        plsc.subcore_barrier()

        def body(k_, carry):
            j0 = k_ * npipe
            ld = []
            for t in range(npipe):
                off = pl.multiple_of((row0 + j0 + t) * _CH, _CH)
                ld.append(pltpu.async_copy(e_hbm.at[pl.ds(off, _CH)], bufs[t], lsem[t]))
            sd = []
            for t in range(npipe):
                ld[t].wait()
                sd.append(pltpu.async_copy(bufs[t], acc.at[di.at[j0 + t]], ssem[t],
                                           add=True))
            for d in sd:
                d.wait()
            return carry

        lax.fori_loop(0, nfull, body, 0)

        for j_ in range(nfull * npipe, nch):
            off = pl.multiple_of((row0 + j_) * _CH, _CH)
            pltpu.sync_copy(e_hbm.at[pl.ds(off, _CH)], bufs[0])
            pltpu.sync_copy(bufs[0], acc.at[di.at[j_]], add=True)

        plsc.subcore_barrier()
        pltpu.sync_copy(acc.at[pl.ds(soff, stripe)],
                        out_hbm.at[pl.ds(pl.multiple_of(c * N + soff, 16), stripe)])

    return k(e2, dst3d, zeros)


def _elu(h):
    return jnp.where(h > 0.0, h, jnp.exp(h) - 1.0)


def _layernorm(h, g, b):
    mu = jnp.mean(h, axis=-1, keepdims=True)
    hc = h - mu
    var = jnp.mean(hc * hc, axis=-1, keepdims=True)
    return hc / jnp.sqrt(var + 1e-5) * g + b


def _edge_mlp(xs, xd, e, ew2, Ws, bs, g, b, E, C):
    BE = 3200

    def body(xs_r, xd_r, e_r, ew_r, w1_r, w2_r, w3_r, w4_r,
             b1_r, b2_r, b3_r, b4_r, g_r, bb_r, out_r):
        w1 = w1_r[...]
        e0 = e_r[...]
        h = (jnp.dot(xs_r[...].astype(jnp.bfloat16), w1[:C],
                     preferred_element_type=jnp.float32)
             + jnp.dot(xd_r[...].astype(jnp.bfloat16), w1[C:2 * C],
                       preferred_element_type=jnp.float32)
             + jnp.dot(e0.astype(jnp.bfloat16), w1[2 * C:],
                       preferred_element_type=jnp.float32)
             + b1_r[...])
        h = _elu(h).astype(jnp.bfloat16)
        h = _elu(jnp.dot(h, w2_r[...], preferred_element_type=jnp.float32)
                 + b2_r[...]).astype(jnp.bfloat16)
        h = _elu(jnp.dot(h, w3_r[...], preferred_element_type=jnp.float32)
                 + b3_r[...]).astype(jnp.bfloat16)
        h = jnp.dot(h, w4_r[...], preferred_element_type=jnp.float32) + b4_r[...]
        out_r[...] = (e0 + _layernorm(h, g_r[...], bb_r[...])) * ew_r[...]

    eb = lambda i: (i, 0)
    zb = lambda i: (0, 0)
    return pl.pallas_call(
        body,
        grid=(E // BE,),
        in_specs=[pl.BlockSpec((BE, C), eb)] * 3
        + [pl.BlockSpec((BE, 1), eb),
           pl.BlockSpec((3 * C, C), zb)]
        + [pl.BlockSpec((C, C), zb)] * 3
        + [pl.BlockSpec((1, C), zb)] * 6,
        out_specs=pl.BlockSpec((BE, C), eb),
        out_shape=jax.ShapeDtypeStruct((E, C), jnp.float32),
    )(xs, xd, e, ew2, Ws[0], Ws[1], Ws[2], Ws[3],
      bs[0], bs[1], bs[2], bs[3], g, b)


def _node_mlp(x, parts, Ws, bs, g, b, N, C):
    BN = 2000

    def body(x_r, p0_r, p1_r, w1_r, w2_r, w3_r, w4_r,
             b1_r, b2_r, b3_r, b4_r, g_r, bb_r, out_r):
        x0 = x_r[...]
        agg = p0_r[...] + p1_r[...]
        w1 = w1_r[...]
        h = (jnp.dot(x0, w1[:C], preferred_element_type=jnp.float32)
             + jnp.dot(agg, w1[C:], preferred_element_type=jnp.float32)
             + b1_r[...])
        h = _elu(h)
        h = _elu(jnp.dot(h, w2_r[...], preferred_element_type=jnp.float32) + b2_r[...])
        h = _elu(jnp.dot(h, w3_r[...], preferred_element_type=jnp.float32) + b3_r[...])
        h = jnp.dot(h, w4_r[...], preferred_element_type=jnp.float32) + b4_r[...]
        out_r[...] = x0 + _layernorm(h, g_r[...], bb_r[...])

    nb = lambda i: (i, 0)
    zb = lambda i: (0, 0)
    p0 = lax.slice_in_dim(parts, 0, N, axis=0)
    p1 = lax.slice_in_dim(parts, N, 2 * N, axis=0)
    return pl.pallas_call(
        body,
        grid=(N // BN,),
        in_specs=[pl.BlockSpec((BN, C), nb)] * 3
        + [pl.BlockSpec((2 * C, C), zb)]
        + [pl.BlockSpec((C, C), zb)] * 3
        + [pl.BlockSpec((1, C), zb)] * 6,
        out_specs=pl.BlockSpec((BN, C), nb),
        out_shape=jax.ShapeDtypeStruct((N, C), jnp.float32),
    )(x, p0, p1, Ws[0], Ws[1], Ws[2], Ws[3],
      bs[0], bs[1], bs[2], bs[3], g, b)


def kernel(x, e, edge_index, edge_weight, halo_info, mask_send, mask_recv,
           buffer_send, buffer_recv, neighboring_procs, SIZE,
           eu_Ws, eu_bs, eu_g, eu_b, nu_Ws, nu_bs, nu_g, nu_b):
    N, C = x.shape
    E = e.shape[0]

    ncg = E // _NW // 40 // 5
    sg3d = edge_index[0].reshape(_NW, 5, ncg, 40)
    dg3d = edge_index[1].reshape(_NW, 5, ncg, 40)
    nch = E // _NW // _CH
    dst3d = edge_index[1].reshape(_NW, nch, _CH)

    xs, xd = _sc_gather(x, sg3d, dg3d, E, N, C)

    e_out = _edge_mlp(
        xs, xd, e, edge_weight.reshape(E, 1),
        [w.astype(jnp.bfloat16) for w in eu_Ws],
        [bb.reshape(1, C) for bb in eu_bs],
        eu_g.reshape(1, C), eu_b.reshape(1, C), E, C)

    zeros = jnp.zeros((640, C), jnp.float32)
    parts = _sc_scatter_add(e_out, dst3d, zeros, E, N, C)

    x_out = _node_mlp(
        x, parts,
        nu_Ws, [bb.reshape(1, C) for bb in nu_bs],
        nu_g.reshape(1, C), nu_b.reshape(1, C), N, C)

    return (x_out, e_out)
